# Initial kernel scaffold; baseline (speedup 1.0000x reference)
import jax
import jax.numpy as jnp
from jax.experimental import pallas as pl


def kernel(x, edge_index, W1, b1, g1, be1, W2, b2, g2, be2, W3, b3, g3, be3):
    raise NotImplementedError("write your pallas kernel here")


# speedup vs baseline: 15.4233x; 15.4233x over previous
import functools

import jax
import jax.numpy as jnp
from jax import lax
from jax.experimental import pallas as pl
from jax.experimental.pallas import tpu as pltpu
from jax.experimental.pallas import tpu_sc as plsc

N = 10000
E = 320000
NPAD = 10240
ROWS_PER_TILE = 640
NW = 32
CHW = 128
NCHUNK = 80
EPAD = NW * NCHUNK * CHW
EPS = 1e-5

_mesh = plsc.VectorSubcoreMesh(core_axis_name="c", subcore_axis_name="s")
_sc_params = pltpu.CompilerParams(use_tc_tiling_on_sc=False)


def _fill_rows(buf, val, d):
    v16 = jnp.full((16,), val, jnp.float32)

    def body(i, carry):
        for cc in range(d // 16):
            buf[i, pl.ds(cc * 16, 16)] = v16
        return carry

    lax.fori_loop(0, CHW, body, 0)


@functools.partial(
    pl.kernel,
    out_type=jax.ShapeDtypeStruct((2, NPAD, 16), jnp.float32),
    mesh=_mesh,
    compiler_params=_sc_params,
    scratch_types=[
        pltpu.VMEM((NCHUNK, CHW), jnp.int32),
        pltpu.VMEM((CHW, 16), jnp.float32),
        pltpu.VMEM_SHARED((NPAD, 16), jnp.float32),
    ],
)
def _sc_degree(dst_hbm, out_hbm, didx, buf, degsh):
    c = lax.axis_index("c")
    s = lax.axis_index("s")
    w = c * 16 + s

    pltpu.sync_copy(dst_hbm.at[w], didx)
    _fill_rows(buf, 0.0, 16)
    for k in range(ROWS_PER_TILE // CHW):
        pltpu.sync_copy(buf, degsh.at[pl.ds(s * ROWS_PER_TILE + k * CHW, CHW)])
    _fill_rows(buf, 1.0, 16)
    plsc.subcore_barrier()

    def body(j, carry):
        pltpu.sync_copy(buf, degsh.at[didx.at[j]], add=True)
        return carry

    lax.fori_loop(0, NCHUNK, body, 0)
    plsc.subcore_barrier()
    pltpu.sync_copy(
        degsh.at[pl.ds(s * ROWS_PER_TILE, ROWS_PER_TILE)],
        out_hbm.at[c].at[pl.ds(s * ROWS_PER_TILE, ROWS_PER_TILE)],
    )


def _make_sc_agg(d):
    @functools.partial(
        pl.kernel,
        out_type=jax.ShapeDtypeStruct((2, NPAD, d), jnp.float32),
        mesh=_mesh,
        compiler_params=_sc_params,
        scratch_types=[
            pltpu.VMEM((NCHUNK, CHW), jnp.int32),
            pltpu.VMEM((NCHUNK, CHW), jnp.int32),
            pltpu.VMEM((CHW, d), jnp.float32),
            pltpu.VMEM((CHW, d), jnp.float32),
            pltpu.VMEM_SHARED((NPAD, d), jnp.float32),
            pltpu.SemaphoreType.DMA,
            pltpu.SemaphoreType.DMA,
        ],
    )
    def sc_agg(hp_hbm, src_hbm, dst_hbm, out_hbm, sidx, didx, rows0, rows1,
               aggsh, sem0, sem1):
        c = lax.axis_index("c")
        s = lax.axis_index("s")
        w = c * 16 + s

        pltpu.sync_copy(src_hbm.at[w], sidx)
        pltpu.sync_copy(dst_hbm.at[w], didx)
        _fill_rows(rows0, 0.0, d)
        for k in range(ROWS_PER_TILE // CHW):
            pltpu.sync_copy(
                rows0, aggsh.at[pl.ds(s * ROWS_PER_TILE + k * CHW, CHW)])
        plsc.subcore_barrier()

        def body(j, carry):
            j0 = 2 * j
            j1 = j0 + 1
            cp0 = pltpu.async_copy(hp_hbm.at[sidx.at[j0]], rows0, sem0)
            cp1 = pltpu.async_copy(hp_hbm.at[sidx.at[j1]], rows1, sem1)
            cp0.wait()
            pltpu.sync_copy(rows0, aggsh.at[didx.at[j0]], add=True)
            cp1.wait()
            pltpu.sync_copy(rows1, aggsh.at[didx.at[j1]], add=True)
            return carry

        lax.fori_loop(0, NCHUNK // 2, body, 0)
        plsc.subcore_barrier()
        pltpu.sync_copy(
            aggsh.at[pl.ds(s * ROWS_PER_TILE, ROWS_PER_TILE)],
            out_hbm.at[c].at[pl.ds(s * ROWS_PER_TILE, ROWS_PER_TILE)],
        )

    return sc_agg


_sc_agg64 = _make_sc_agg(64)
_sc_agg16 = _make_sc_agg(16)


def _dinv_from(degp_ref):
    deg = degp_ref[0, :N, 0:1] + degp_ref[1, :N, 0:1] + 1.0
    return lax.rsqrt(deg)


def _tc_pre(x, w1, degp):
    def body(x_ref, w_ref, degp_ref, out_ref):
        dinv = _dinv_from(degp_ref)
        out_ref[...] = jnp.dot(x_ref[...] * dinv, w_ref[...],
                               preferred_element_type=jnp.float32)

    return pl.pallas_call(
        body, out_shape=jax.ShapeDtypeStruct((N, w1.shape[1]), jnp.float32),
    )(x, w1, degp)


def _tc_mid(aggp, hp, degp, g, be, wn):
    def body(aggp_ref, hp_ref, degp_ref, g_ref, be_ref, w_ref, out_ref):
        dinv = _dinv_from(degp_ref)
        conv = dinv * (aggp_ref[0, :N] + aggp_ref[1, :N] + hp_ref[...])
        mu = jnp.mean(conv, axis=0, keepdims=True)
        xc = conv - mu
        var = jnp.mean(xc * xc, axis=0, keepdims=True)
        h = g_ref[...] * xc * lax.rsqrt(var + EPS) + be_ref[...]
        h = jnp.maximum(h, 0.0) * dinv
        out_ref[...] = jnp.dot(h, w_ref[...],
                               preferred_element_type=jnp.float32)

    return pl.pallas_call(
        body, out_shape=jax.ShapeDtypeStruct((N, wn.shape[1]), jnp.float32),
    )(aggp, hp, degp, g, be, wn)


def _tc_post(aggp, hp, degp, g, be):
    def body(aggp_ref, hp_ref, degp_ref, g_ref, be_ref, out_ref):
        dinv = _dinv_from(degp_ref)
        conv = dinv * (aggp_ref[0, :N] + aggp_ref[1, :N] + hp_ref[...])
        mu = jnp.mean(conv, axis=0, keepdims=True)
        xc = conv - mu
        var = jnp.mean(xc * xc, axis=0, keepdims=True)
        out_ref[...] = g_ref[...] * xc * lax.rsqrt(var + EPS) + be_ref[...]

    return pl.pallas_call(
        body, out_shape=jax.ShapeDtypeStruct((N, 16), jnp.float32),
    )(aggp, hp, degp, g, be)


def kernel(x, edge_index, W1, b1, g1, be1, W2, b2, g2, be2, W3, b3, g3, be3):
    del b1, b2, b3
    src = edge_index[0].astype(jnp.int32)
    dst = edge_index[1].astype(jnp.int32)
    pad = EPAD - E
    src_t = jnp.concatenate([src, jnp.zeros((pad,), jnp.int32)]
                            ).reshape(NW, NCHUNK, CHW)
    dst_t = jnp.concatenate([dst, jnp.full((pad,), N, jnp.int32)]
                            ).reshape(NW, NCHUNK, CHW)

    degp = _sc_degree(dst_t)

    g1r = g1.reshape(1, -1)
    be1r = be1.reshape(1, -1)
    g2r = g2.reshape(1, -1)
    be2r = be2.reshape(1, -1)
    g3p = jnp.concatenate([g3, jnp.ones((16 - g3.shape[0],), jnp.float32)]
                          ).reshape(1, 16)
    be3p = jnp.concatenate([be3, jnp.zeros((16 - be3.shape[0],), jnp.float32)]
                           ).reshape(1, 16)
    w3p = jnp.concatenate(
        [W3, jnp.zeros((W3.shape[0], 16 - W3.shape[1]), jnp.float32)], axis=1)

    hp1 = _tc_pre(x, W1, degp)
    agg1 = _sc_agg64(hp1, src_t, dst_t)
    hp2 = _tc_mid(agg1, hp1, degp, g1r, be1r, W2)
    agg2 = _sc_agg64(hp2, src_t, dst_t)
    hp3 = _tc_mid(agg2, hp2, degp, g2r, be2r, w3p)
    agg3 = _sc_agg16(hp3, src_t, dst_t)
    out = _tc_post(agg3, hp3, degp, g3p, be3p)
    return out[:, :W3.shape[1]]


# speedup vs baseline: 16.9493x; 1.0989x over previous
import functools

import jax
import jax.numpy as jnp
from jax import lax
from jax.experimental import pallas as pl
from jax.experimental.pallas import tpu as pltpu
from jax.experimental.pallas import tpu_sc as plsc

N = 10000
E = 320000
NPAD = 10240
ROWS_PER_TILE = 640
NW = 32
CHW = 128
NCHUNK = 80
EPAD = NW * NCHUNK * CHW
EPS = 1e-5

_mesh = plsc.VectorSubcoreMesh(core_axis_name="c", subcore_axis_name="s")
_sc_params = pltpu.CompilerParams(use_tc_tiling_on_sc=False)


def _fill_rows(buf, val, d):
    v16 = jnp.full((16,), val, jnp.float32)

    def body(i, carry):
        for cc in range(d // 16):
            buf[i, pl.ds(cc * 16, 16)] = v16
        return carry

    lax.fori_loop(0, CHW, body, 0)


@functools.partial(
    pl.kernel,
    out_type=jax.ShapeDtypeStruct((2, NPAD, 16), jnp.float32),
    mesh=_mesh,
    compiler_params=_sc_params,
    scratch_types=[
        pltpu.VMEM((NCHUNK, CHW), jnp.int32),
        pltpu.VMEM((CHW, 16), jnp.float32),
        pltpu.VMEM_SHARED((NPAD, 16), jnp.float32),
        pltpu.SemaphoreType.DMA,
    ],
)
def _sc_degree(dst_hbm, out_hbm, didx, buf, degsh, sem):
    c = lax.axis_index("c")
    s = lax.axis_index("s")
    w = c * 16 + s

    pltpu.sync_copy(dst_hbm.at[w], didx)
    _fill_rows(buf, 0.0, 16)
    for k in range(ROWS_PER_TILE // CHW):
        pltpu.sync_copy(buf, degsh.at[pl.ds(s * ROWS_PER_TILE + k * CHW, CHW)])
    _fill_rows(buf, 1.0, 16)
    plsc.subcore_barrier()

    def fire(j, carry):
        pltpu.async_copy(buf, degsh.at[didx.at[j]], sem, add=True)
        return carry

    def drain(j, carry):
        pltpu.make_async_copy(buf, degsh.at[didx.at[j]], sem).wait()
        return carry

    lax.fori_loop(0, NCHUNK, fire, 0)
    lax.fori_loop(0, NCHUNK, drain, 0)
    plsc.subcore_barrier()
    pltpu.sync_copy(
        degsh.at[pl.ds(s * ROWS_PER_TILE, ROWS_PER_TILE)],
        out_hbm.at[c].at[pl.ds(s * ROWS_PER_TILE, ROWS_PER_TILE)],
    )


def _make_sc_agg(d):
    K = 4
    NG = NCHUNK // K

    @functools.partial(
        pl.kernel,
        out_type=jax.ShapeDtypeStruct((2, NPAD, d), jnp.float32),
        mesh=_mesh,
        compiler_params=_sc_params,
        scratch_types=[
            pltpu.VMEM((NCHUNK, CHW), jnp.int32),
            pltpu.VMEM((NCHUNK, CHW), jnp.int32),
            pltpu.VMEM((2, K, CHW, d), jnp.float32),
            pltpu.VMEM_SHARED((NPAD, d), jnp.float32),
            pltpu.SemaphoreType.DMA,
            pltpu.SemaphoreType.DMA,
            pltpu.SemaphoreType.DMA,
            pltpu.SemaphoreType.DMA,
        ],
    )
    def sc_agg(hp_hbm, src_hbm, dst_hbm, out_hbm, sidx, didx, rows,
               aggsh, gsem0, gsem1, ssem0, ssem1):
        c = lax.axis_index("c")
        s = lax.axis_index("s")
        w = c * 16 + s
        gsem = (gsem0, gsem1)
        ssem = (ssem0, ssem1)

        pltpu.sync_copy(src_hbm.at[w], sidx)
        pltpu.sync_copy(dst_hbm.at[w], didx)
        _fill_rows(rows.at[0, 0], 0.0, d)
        for k in range(ROWS_PER_TILE // CHW):
            pltpu.sync_copy(
                rows.at[0, 0], aggsh.at[pl.ds(s * ROWS_PER_TILE + k * CHW, CHW)])
        plsc.subcore_barrier()

        def fire_g(p, b, j):
            pltpu.async_copy(hp_hbm.at[sidx.at[j]], rows.at[p, b], gsem[p])

        def wait_g(p, b, j):
            pltpu.make_async_copy(
                hp_hbm.at[sidx.at[j]], rows.at[p, b], gsem[p]).wait()

        def fire_s(p, b, j):
            pltpu.async_copy(rows.at[p, b], aggsh.at[didx.at[j]], ssem[p],
                             add=True)

        def wait_s(p, b, j):
            pltpu.make_async_copy(
                rows.at[p, b], aggsh.at[didx.at[j]], ssem[p]).wait()

        for b in range(K):
            fire_g(0, b, b)

        def body(t2, carry):
            g0 = 2 * t2
            g1 = g0 + 1
            for b in range(K):
                wait_g(0, b, g0 * K + b)

            @pl.when(t2 > 0)
            def _():
                for b in range(K):
                    wait_s(1, b, (g0 - 1) * K + b)

            for b in range(K):
                fire_g(1, b, g1 * K + b)
            for b in range(K):
                fire_s(0, b, g0 * K + b)
            for b in range(K):
                wait_g(1, b, g1 * K + b)
            for b in range(K):
                wait_s(0, b, g0 * K + b)

            @pl.when(g1 + 1 < NG)
            def _():
                for b in range(K):
                    fire_g(0, b, (g1 + 1) * K + b)

            for b in range(K):
                fire_s(1, b, g1 * K + b)
            return carry

        lax.fori_loop(0, NG // 2, body, 0)
        for b in range(K):
            wait_s(1, b, (NG - 1) * K + b)
        plsc.subcore_barrier()
        pltpu.sync_copy(
            aggsh.at[pl.ds(s * ROWS_PER_TILE, ROWS_PER_TILE)],
            out_hbm.at[c].at[pl.ds(s * ROWS_PER_TILE, ROWS_PER_TILE)],
        )

    return sc_agg


_sc_agg64 = _make_sc_agg(64)
_sc_agg16 = _make_sc_agg(16)


def _dinv_from(degp_ref):
    deg = degp_ref[0, :N, 0:1] + degp_ref[1, :N, 0:1] + 1.0
    return lax.rsqrt(deg)


def _tc_pre(x, w1, degp):
    def body(x_ref, w_ref, degp_ref, out_ref):
        dinv = _dinv_from(degp_ref)
        out_ref[...] = jnp.dot(x_ref[...] * dinv, w_ref[...],
                               preferred_element_type=jnp.float32)

    return pl.pallas_call(
        body, out_shape=jax.ShapeDtypeStruct((N, w1.shape[1]), jnp.float32),
    )(x, w1, degp)


def _tc_mid(aggp, hp, degp, g, be, wn):
    def body(aggp_ref, hp_ref, degp_ref, g_ref, be_ref, w_ref, out_ref):
        dinv = _dinv_from(degp_ref)
        conv = dinv * (aggp_ref[0, :N] + aggp_ref[1, :N] + hp_ref[...])
        mu = jnp.mean(conv, axis=0, keepdims=True)
        xc = conv - mu
        var = jnp.mean(xc * xc, axis=0, keepdims=True)
        h = g_ref[...] * xc * lax.rsqrt(var + EPS) + be_ref[...]
        h = jnp.maximum(h, 0.0) * dinv
        out_ref[...] = jnp.dot(h, w_ref[...],
                               preferred_element_type=jnp.float32)

    return pl.pallas_call(
        body, out_shape=jax.ShapeDtypeStruct((N, wn.shape[1]), jnp.float32),
    )(aggp, hp, degp, g, be, wn)


def _tc_post(aggp, hp, degp, g, be):
    def body(aggp_ref, hp_ref, degp_ref, g_ref, be_ref, out_ref):
        dinv = _dinv_from(degp_ref)
        conv = dinv * (aggp_ref[0, :N] + aggp_ref[1, :N] + hp_ref[...])
        mu = jnp.mean(conv, axis=0, keepdims=True)
        xc = conv - mu
        var = jnp.mean(xc * xc, axis=0, keepdims=True)
        out_ref[...] = g_ref[...] * xc * lax.rsqrt(var + EPS) + be_ref[...]

    return pl.pallas_call(
        body, out_shape=jax.ShapeDtypeStruct((N, 16), jnp.float32),
    )(aggp, hp, degp, g, be)


def kernel(x, edge_index, W1, b1, g1, be1, W2, b2, g2, be2, W3, b3, g3, be3):
    del b1, b2, b3
    src = edge_index[0].astype(jnp.int32)
    dst = edge_index[1].astype(jnp.int32)
    pad = EPAD - E
    src_t = jnp.concatenate([src, jnp.zeros((pad,), jnp.int32)]
                            ).reshape(NW, NCHUNK, CHW)
    dst_t = jnp.concatenate([dst, jnp.full((pad,), N, jnp.int32)]
                            ).reshape(NW, NCHUNK, CHW)

    degp = _sc_degree(dst_t)

    g1r = g1.reshape(1, -1)
    be1r = be1.reshape(1, -1)
    g2r = g2.reshape(1, -1)
    be2r = be2.reshape(1, -1)
    g3p = jnp.concatenate([g3, jnp.ones((16 - g3.shape[0],), jnp.float32)]
                          ).reshape(1, 16)
    be3p = jnp.concatenate([be3, jnp.zeros((16 - be3.shape[0],), jnp.float32)]
                           ).reshape(1, 16)
    w3p = jnp.concatenate(
        [W3, jnp.zeros((W3.shape[0], 16 - W3.shape[1]), jnp.float32)], axis=1)

    hp1 = _tc_pre(x, W1, degp)
    agg1 = _sc_agg64(hp1, src_t, dst_t)
    hp2 = _tc_mid(agg1, hp1, degp, g1r, be1r, W2)
    agg2 = _sc_agg64(hp2, src_t, dst_t)
    hp3 = _tc_mid(agg2, hp2, degp, g2r, be2r, w3p)
    agg3 = _sc_agg16(hp3, src_t, dst_t)
    out = _tc_post(agg3, hp3, degp, g3p, be3p)
    return out[:, :W3.shape[1]]


# speedup vs baseline: 18.1782x; 1.0725x over previous
import functools

import jax
import jax.numpy as jnp
from jax import lax
from jax.experimental import pallas as pl
from jax.experimental.pallas import tpu as pltpu
from jax.experimental.pallas import tpu_sc as plsc

N = 10000
E = 320000
NPAD = 10240
ROWS_PER_TILE = 640
NW = 32
CHW = 128
NCHUNK = 80
EPAD = NW * NCHUNK * CHW
EPS = 1e-5

_mesh = plsc.VectorSubcoreMesh(core_axis_name="c", subcore_axis_name="s")
_sc_params = pltpu.CompilerParams(use_tc_tiling_on_sc=False)


def _fill_rows(buf, val, d):
    v16 = jnp.full((16,), val, jnp.float32)

    def body(i, carry):
        for cc in range(d // 16):
            buf[i, pl.ds(cc * 16, 16)] = v16
        return carry

    lax.fori_loop(0, CHW, body, 0)


@functools.partial(
    pl.kernel,
    out_type=jax.ShapeDtypeStruct((2, NPAD, 16), jnp.float32),
    mesh=_mesh,
    compiler_params=_sc_params,
    scratch_types=[
        pltpu.VMEM((NCHUNK, CHW), jnp.int32),
        pltpu.VMEM((CHW, 16), jnp.float32),
        pltpu.VMEM_SHARED((NPAD, 16), jnp.float32),
        pltpu.SemaphoreType.DMA,
    ],
)
def _sc_degree(dst_hbm, out_hbm, didx, buf, degsh, sem):
    c = lax.axis_index("c")
    s = lax.axis_index("s")
    w = c * 16 + s

    pltpu.sync_copy(dst_hbm.at[w], didx)
    _fill_rows(buf, 0.0, 16)
    for k in range(ROWS_PER_TILE // CHW):
        pltpu.sync_copy(buf, degsh.at[pl.ds(s * ROWS_PER_TILE + k * CHW, CHW)])
    _fill_rows(buf, 1.0, 16)
    plsc.subcore_barrier()

    def fire(j, carry):
        pltpu.async_copy(buf, degsh.at[didx.at[j]], sem, add=True)
        return carry

    def drain(j, carry):
        pltpu.make_async_copy(buf, degsh.at[didx.at[j]], sem).wait()
        return carry

    lax.fori_loop(0, NCHUNK, fire, 0)
    lax.fori_loop(0, NCHUNK, drain, 0)
    plsc.subcore_barrier()
    pltpu.sync_copy(
        degsh.at[pl.ds(s * ROWS_PER_TILE, ROWS_PER_TILE)],
        out_hbm.at[c].at[pl.ds(s * ROWS_PER_TILE, ROWS_PER_TILE)],
    )


def _make_sc_agg(d, variant="full"):
    K = 4
    NG = NCHUNK // K

    @functools.partial(
        pl.kernel,
        out_type=jax.ShapeDtypeStruct((2, NPAD, d), jnp.float32),
        mesh=_mesh,
        compiler_params=_sc_params,
        scratch_types=[
            pltpu.VMEM((NCHUNK, CHW), jnp.int32),
            pltpu.VMEM((NCHUNK, CHW), jnp.int32),
            pltpu.VMEM((2, K, CHW, d), jnp.float32),
            pltpu.VMEM_SHARED((NPAD, d), jnp.float32),
            pltpu.SemaphoreType.DMA,
            pltpu.SemaphoreType.DMA,
            pltpu.SemaphoreType.DMA,
            pltpu.SemaphoreType.DMA,
        ],
    )
    def sc_agg(hp_hbm, src_hbm, dst_hbm, out_hbm, sidx, didx, rows,
               aggsh, gsem0, gsem1, ssem0, ssem1):
        c = lax.axis_index("c")
        s = lax.axis_index("s")
        w = c * 16 + s
        gsem = (gsem0, gsem1)
        ssem = (ssem0, ssem1)

        pltpu.sync_copy(src_hbm.at[w], sidx)
        pltpu.sync_copy(dst_hbm.at[w], didx)
        _fill_rows(rows.at[0, 0], 0.0, d)
        for k in range(ROWS_PER_TILE // CHW):
            pltpu.sync_copy(
                rows.at[0, 0], aggsh.at[pl.ds(s * ROWS_PER_TILE + k * CHW, CHW)])
        plsc.subcore_barrier()

        def fire_g(p, b, j):
            pltpu.async_copy(hp_hbm.at[sidx.at[j]], rows.at[p, b], gsem[p])

        def wait_g(p, b, j):
            pltpu.make_async_copy(
                hp_hbm.at[sidx.at[j]], rows.at[p, b], gsem[p]).wait()

        def fire_s(p, b, j):
            pltpu.async_copy(rows.at[p, b], aggsh.at[didx.at[j]], ssem[p],
                             add=True)

        def wait_s(p, b, j):
            pltpu.make_async_copy(
                rows.at[p, b], aggsh.at[didx.at[j]], ssem[p]).wait()

        for b in range(K):
            fire_g(0, b, b)

        def body(t2, carry):
            g0 = 2 * t2
            g1 = g0 + 1
            for b in range(K):
                wait_g(0, b, g0 * K + b)

            @pl.when(t2 > 0)
            def _():
                for b in range(K):
                    wait_s(1, b, (g0 - 1) * K + b)

            for b in range(K):
                fire_g(1, b, g1 * K + b)
            for b in range(K):
                fire_s(0, b, g0 * K + b)
            for b in range(K):
                wait_g(1, b, g1 * K + b)
            for b in range(K):
                wait_s(0, b, g0 * K + b)

            @pl.when(g1 + 1 < NG)
            def _():
                for b in range(K):
                    fire_g(0, b, (g1 + 1) * K + b)

            for b in range(K):
                fire_s(1, b, g1 * K + b)
            return carry

        lax.fori_loop(0, NG // 2, body, 0)
        for b in range(K):
            wait_s(1, b, (NG - 1) * K + b)
        plsc.subcore_barrier()
        pltpu.sync_copy(
            aggsh.at[pl.ds(s * ROWS_PER_TILE, ROWS_PER_TILE)],
            out_hbm.at[c].at[pl.ds(s * ROWS_PER_TILE, ROWS_PER_TILE)],
        )

    return sc_agg


def _make_sc_agg_ring(d, rd=8, rl=2):
    @functools.partial(
        pl.kernel,
        out_type=jax.ShapeDtypeStruct((2, NPAD, d), jnp.float32),
        mesh=_mesh,
        compiler_params=_sc_params,
        scratch_types=[
            pltpu.VMEM((NCHUNK, CHW), jnp.int32),
            pltpu.VMEM((NCHUNK, CHW), jnp.int32),
            pltpu.VMEM((rd, CHW, d), jnp.float32),
            pltpu.VMEM_SHARED((NPAD, d), jnp.float32),
            pltpu.SemaphoreType.DMA,
            pltpu.SemaphoreType.DMA,
        ],
    )
    def sc_agg(hp_hbm, src_hbm, dst_hbm, out_hbm, sidx, didx, rows,
               aggsh, gsem, ssem):
        c = lax.axis_index("c")
        s = lax.axis_index("s")
        w = c * 16 + s

        pltpu.sync_copy(src_hbm.at[w], sidx)
        pltpu.sync_copy(dst_hbm.at[w], didx)
        _fill_rows(rows.at[0], 0.0, d)
        for k in range(ROWS_PER_TILE // CHW):
            pltpu.sync_copy(
                rows.at[0], aggsh.at[pl.ds(s * ROWS_PER_TILE + k * CHW, CHW)])
        plsc.subcore_barrier()

        def fire_g(b, j):
            pltpu.async_copy(hp_hbm.at[sidx.at[j]], rows.at[b], gsem)

        def wait_g(b, j):
            pltpu.make_async_copy(
                hp_hbm.at[sidx.at[j]], rows.at[b], gsem).wait()

        def fire_s(b, j):
            pltpu.async_copy(rows.at[b], aggsh.at[didx.at[j]], ssem, add=True)

        def wait_s(b, j):
            pltpu.make_async_copy(
                rows.at[b], aggsh.at[didx.at[j]], ssem).wait()

        for j in range(rd - rl):
            fire_g(j % rd, j)

        def body(t2, carry):
            for b in range(rd):
                t = t2 * rd + b

                @pl.when(t >= rl)
                def _():
                    wait_s((t - rl) % rd, t - rl)

                @pl.when(t < NCHUNK - rd + rl)
                def _():
                    fire_g((t - rl) % rd, t - rl + rd)

                wait_g(b, t)
                fire_s(b, t)
            return carry

        lax.fori_loop(0, NCHUNK // rd, body, 0)
        for j in range(NCHUNK - rl, NCHUNK):
            wait_s(j % rd, j)
        plsc.subcore_barrier()
        pltpu.sync_copy(
            aggsh.at[pl.ds(s * ROWS_PER_TILE, ROWS_PER_TILE)],
            out_hbm.at[c].at[pl.ds(s * ROWS_PER_TILE, ROWS_PER_TILE)],
        )

    return sc_agg


_sc_agg64 = _make_sc_agg_ring(64)
_sc_agg64b = _make_sc_agg(64)
_sc_agg16 = _make_sc_agg(16)


def _dinv_from(degp_ref):
    deg = degp_ref[0, :N, 0:1] + degp_ref[1, :N, 0:1] + 1.0
    return lax.rsqrt(deg)


def _tc_mm(x, w1):
    def body(x_ref, w_ref, out_ref):
        out_ref[...] = jnp.dot(x_ref[...], w_ref[...],
                               preferred_element_type=jnp.float32)

    return pl.pallas_call(
        body, out_shape=jax.ShapeDtypeStruct((N, w1.shape[1]), jnp.float32),
    )(x, w1)


def _tc_scale(xw, degp):
    def body(xw_ref, degp_ref, out_ref):
        out_ref[...] = xw_ref[...] * _dinv_from(degp_ref)

    return pl.pallas_call(
        body, out_shape=jax.ShapeDtypeStruct(xw.shape, jnp.float32),
    )(xw, degp)


def _tc_mid(aggp, hp, degp, g, be, wn):
    def body(aggp_ref, hp_ref, degp_ref, g_ref, be_ref, w_ref, out_ref):
        dinv = _dinv_from(degp_ref)
        conv = dinv * (aggp_ref[0, :N] + aggp_ref[1, :N] + hp_ref[...])
        mu = jnp.mean(conv, axis=0, keepdims=True)
        xc = conv - mu
        var = jnp.mean(xc * xc, axis=0, keepdims=True)
        h = g_ref[...] * xc * lax.rsqrt(var + EPS) + be_ref[...]
        h = jnp.maximum(h, 0.0) * dinv
        out_ref[...] = jnp.dot(h, w_ref[...],
                               preferred_element_type=jnp.float32)

    return pl.pallas_call(
        body, out_shape=jax.ShapeDtypeStruct((N, wn.shape[1]), jnp.float32),
    )(aggp, hp, degp, g, be, wn)


def _tc_post(aggp, hp, degp, g, be):
    def body(aggp_ref, hp_ref, degp_ref, g_ref, be_ref, out_ref):
        dinv = _dinv_from(degp_ref)
        conv = dinv * (aggp_ref[0, :N] + aggp_ref[1, :N] + hp_ref[...])
        mu = jnp.mean(conv, axis=0, keepdims=True)
        xc = conv - mu
        var = jnp.mean(xc * xc, axis=0, keepdims=True)
        out_ref[...] = g_ref[...] * xc * lax.rsqrt(var + EPS) + be_ref[...]

    return pl.pallas_call(
        body, out_shape=jax.ShapeDtypeStruct((N, 16), jnp.float32),
    )(aggp, hp, degp, g, be)


def kernel(x, edge_index, W1, b1, g1, be1, W2, b2, g2, be2, W3, b3, g3, be3):
    del b1, b2, b3
    src = edge_index[0].astype(jnp.int32)
    dst = edge_index[1].astype(jnp.int32)
    pad = EPAD - E
    src_t = jnp.concatenate([src, jnp.zeros((pad,), jnp.int32)]
                            ).reshape(NW, NCHUNK, CHW)
    dst_t = jnp.concatenate([dst, jnp.full((pad,), N, jnp.int32)]
                            ).reshape(NW, NCHUNK, CHW)

    degp = _sc_degree(dst_t)

    g1r = g1.reshape(1, -1)
    be1r = be1.reshape(1, -1)
    g2r = g2.reshape(1, -1)
    be2r = be2.reshape(1, -1)
    g3p = jnp.concatenate([g3, jnp.ones((16 - g3.shape[0],), jnp.float32)]
                          ).reshape(1, 16)
    be3p = jnp.concatenate([be3, jnp.zeros((16 - be3.shape[0],), jnp.float32)]
                           ).reshape(1, 16)
    w3p = jnp.concatenate(
        [W3, jnp.zeros((W3.shape[0], 16 - W3.shape[1]), jnp.float32)], axis=1)

    hp1 = _tc_scale(_tc_mm(x, W1), degp)
    agg1 = _sc_agg64(hp1, src_t, dst_t)
    hp2 = _tc_mid(agg1, hp1, degp, g1r, be1r, W2)
    agg2 = _sc_agg64b(hp2, src_t, dst_t)
    hp3 = _tc_mid(agg2, hp2, degp, g2r, be2r, w3p)
    agg3 = _sc_agg16(hp3, src_t, dst_t)
    out = _tc_post(agg3, hp3, degp, g3p, be3p)
    return out[:, :W3.shape[1]]
